# baseline (speedup 1.0000x reference)
import functools

import jax
import jax.numpy as jnp
import numpy as np
from jax import lax
from jax.experimental import pallas as pl
from jax.experimental.pallas import tpu as pltpu
from jax.experimental.pallas import tpu_sc as plsc

N_DIM = 16
LANES = 16
CB = 128


def _mf_kernel(b_per_w, num_cores, notes_hbm, raters_hbm, note_emb_hbm,
               rater_emb_hbm, note_bias_hbm, rater_bias_hbm, rater_rep_hbm,
               gb_hbm, out_hbm, idx_n, idx_r, qn, qr, ne_rows, re_rows,
               nb_rows, rb_rows, rr_rows, gb_v, out_v, sem):
    wid = lax.axis_index("s") * num_cores + lax.axis_index("c")
    n_chunks = b_per_w // CB
    row_base = wid * n_chunks

    pltpu.sync_copy(notes_hbm.at[pl.ds(row_base, n_chunks)], idx_n)
    pltpu.sync_copy(raters_hbm.at[pl.ds(row_base, n_chunks)], idx_r)
    pltpu.sync_copy(gb_hbm, gb_v)

    gb = gb_v[...]
    inv_sqrt_dim = np.float32(1.0 / np.sqrt(N_DIM))
    one = jnp.float32(1.0)
    fifteen = jnp.full((LANES,), 15, jnp.int32)
    iota = lax.iota(jnp.int32, LANES)

    for c in range(n_chunks):
        for g in range(CB // LANES):
            s = pl.ds(g * LANES, LANES)
            qn[s] = lax.shift_right_logical(idx_n[c, s], 4)
            qr[s] = lax.shift_right_logical(idx_r[c, s], 4)

        copies = []
        for d in range(N_DIM):
            copies.append(pltpu.make_async_copy(
                note_emb_hbm.at[d].at[qn], ne_rows.at[d], sem))
            copies.append(pltpu.make_async_copy(
                rater_emb_hbm.at[d].at[qr], re_rows.at[d], sem))
        copies.append(pltpu.make_async_copy(note_bias_hbm.at[qn], nb_rows, sem))
        copies.append(pltpu.make_async_copy(rater_bias_hbm.at[qr], rb_rows, sem))
        copies.append(pltpu.make_async_copy(rater_rep_hbm.at[qr], rr_rows, sem))
        for cp in copies:
            cp.start()
        for cp in copies:
            cp.wait()

        for g in range(CB // LANES):
            s = pl.ds(g * LANES, LANES)
            rows = jnp.full((LANES,), g * LANES, jnp.int32) + iota
            ncol = lax.bitwise_and(idx_n[c, s], fifteen)
            rcol = lax.bitwise_and(idx_r[c, s], fifteen)
            d0 = jnp.zeros((LANES,), jnp.int32)
            acc = (plsc.load_gather(ne_rows, [d0, rows, ncol])
                   * plsc.load_gather(re_rows, [d0, rows, rcol]))
            for d in range(1, N_DIM):
                dv = jnp.full((LANES,), d, jnp.int32)
                acc = acc + (plsc.load_gather(ne_rows, [dv, rows, ncol])
                             * plsc.load_gather(re_rows, [dv, rows, rcol]))
            nb = plsc.load_gather(nb_rows, [rows, ncol])
            rb = plsc.load_gather(rb_rows, [rows, rcol])
            rr = plsc.load_gather(rr_rows, [rows, rcol])
            pred = acc * inv_sqrt_dim + nb * rr + rb + gb
            out_v[pl.ds(c * CB + g * LANES, LANES)] = one / (one + jnp.exp(-pred))

    pltpu.sync_copy(out_v, out_hbm.at[pl.ds(wid * b_per_w, b_per_w)])


def kernel(notes, raters, noteEmb, raterEmb, noteBias, raterBias, raterRep,
           globalBias):
    batch = notes.shape[0]
    n_notes = noteEmb.shape[0]
    n_raters = raterEmb.shape[0]
    info = plsc.get_sparse_core_info()
    num_workers = info.num_cores * info.num_subcores
    b_per_w = batch // num_workers

    notes2d = notes.astype(jnp.int32).reshape(batch // CB, CB)
    raters2d = raters.astype(jnp.int32).reshape(batch // CB, CB)
    neT = noteEmb.T.reshape(N_DIM, n_notes // LANES, LANES)
    reT = raterEmb.T.reshape(N_DIM, n_raters // LANES, LANES)
    nb2 = noteBias.reshape(-1).reshape(n_notes // LANES, LANES)
    rb2 = raterBias.reshape(-1).reshape(n_raters // LANES, LANES)
    rr2 = raterRep.reshape(-1).reshape(n_raters // LANES, LANES)
    gb16 = jnp.broadcast_to(globalBias.astype(jnp.float32), (LANES,))

    mesh = plsc.VectorSubcoreMesh(core_axis_name="c", subcore_axis_name="s")
    run = pl.kernel(
        functools.partial(_mf_kernel, b_per_w, info.num_cores),
        out_type=jax.ShapeDtypeStruct((batch,), jnp.float32),
        mesh=mesh,
        compiler_params=pltpu.CompilerParams(
            needs_layout_passes=False, use_tc_tiling_on_sc=False),
        scratch_types=[
            pltpu.VMEM((b_per_w // CB, CB), jnp.int32),
            pltpu.VMEM((b_per_w // CB, CB), jnp.int32),
            pltpu.VMEM((CB,), jnp.int32),
            pltpu.VMEM((CB,), jnp.int32),
            pltpu.VMEM((N_DIM, CB, LANES), jnp.float32),
            pltpu.VMEM((N_DIM, CB, LANES), jnp.float32),
            pltpu.VMEM((CB, LANES), jnp.float32),
            pltpu.VMEM((CB, LANES), jnp.float32),
            pltpu.VMEM((CB, LANES), jnp.float32),
            pltpu.VMEM((LANES,), jnp.float32),
            pltpu.VMEM((b_per_w,), jnp.float32),
            pltpu.SemaphoreType.DMA,
        ],
    )
    out = run(notes2d, raters2d, neT, reT, nb2, rb2, rr2, gb16)
    return out.reshape(batch, 1)


# speedup vs baseline: 2.6948x; 2.6948x over previous
import functools

import jax
import jax.numpy as jnp
import numpy as np
from jax import lax
from jax.experimental import pallas as pl
from jax.experimental.pallas import tpu as pltpu
from jax.experimental.pallas import tpu_sc as plsc

N_DIM = 16
LANES = 16
CB = 128


def _mf_kernel(b_per_w, num_cores, notes_hbm, raters_hbm, ne_lin, re_lin,
               nb2, rb2, rr2, gb_hbm, out_hbm, idx_n, idx_r, qn, qr, qbn, qbr,
               nrows, rrows, nbrows, rbrows, rrrows, gb_v, out_v, sem):
    sid = lax.axis_index("s")
    wid = sid * num_cores + lax.axis_index("c")
    n_chunks = b_per_w // CB
    row_base = wid * n_chunks

    pltpu.sync_copy(notes_hbm.at[pl.ds(row_base, n_chunks)], idx_n)
    pltpu.sync_copy(raters_hbm.at[pl.ds(row_base, n_chunks)], idx_r)
    pltpu.sync_copy(gb_hbm, gb_v)

    gb = gb_v[...]
    inv_sqrt_dim = np.float32(1.0 / np.sqrt(N_DIM))
    one = jnp.float32(1.0)
    seven = jnp.full((LANES,), 7, jnp.int32)
    c127 = jnp.full((LANES,), 127, jnp.int32)
    three = jnp.full((LANES,), 3, jnp.int32)
    sb7 = jnp.full((LANES,), 7, jnp.int32)
    four = jnp.full((LANES,), 4, jnp.int32)
    iota = lax.iota(jnp.int32, LANES)

    for c in range(n_chunks):
        for g in range(CB // LANES):
            s = pl.ds(g * LANES, LANES)
            nidx = idx_n[c, s]
            ridx = idx_r[c, s]
            qn[s] = lax.shift_right_logical(nidx, three)
            qr[s] = lax.shift_right_logical(ridx, three)
            qbn[s] = lax.shift_right_logical(nidx, sb7)
            qbr[s] = lax.shift_right_logical(ridx, sb7)

        copies = [
            pltpu.make_async_copy(ne_lin.at[qn], nrows, sem),
            pltpu.make_async_copy(re_lin.at[qr], rrows, sem),
            pltpu.make_async_copy(nb2.at[qbn], nbrows, sem),
            pltpu.make_async_copy(rb2.at[qbr], rbrows, sem),
            pltpu.make_async_copy(rr2.at[qbr], rrrows, sem),
        ]
        for cp in copies:
            cp.start()
        for cp in copies:
            cp.wait()

        for g in range(CB // LANES):
            s = pl.ds(g * LANES, LANES)
            nidx = idx_n[c, s]
            ridx = idx_r[c, s]
            rows = jnp.full((LANES,), g * LANES, jnp.int32) + iota
            ncol = lax.shift_left(lax.bitwise_and(nidx, seven), four)
            rcol = lax.shift_left(lax.bitwise_and(ridx, seven), four)
            acc = (plsc.load_gather(nrows, [rows, ncol])
                   * plsc.load_gather(rrows, [rows, rcol]))
            for d in range(1, N_DIM):
                dv = jnp.full((LANES,), d, jnp.int32)
                acc = acc + (plsc.load_gather(nrows, [rows, ncol + dv])
                             * plsc.load_gather(rrows, [rows, rcol + dv]))
            nbcol = lax.bitwise_and(nidx, c127)
            rbcol = lax.bitwise_and(ridx, c127)
            nb = plsc.load_gather(nbrows, [rows, nbcol])
            rb = plsc.load_gather(rbrows, [rows, rbcol])
            rr = plsc.load_gather(rrrows, [rows, rbcol])
            pred = acc * inv_sqrt_dim + nb * rr + rb + gb
            out_v[pl.ds(c * CB + g * LANES, LANES)] = one / (one + jnp.exp(-pred))

    pltpu.sync_copy(out_v, out_hbm.at[pl.ds(wid * b_per_w, b_per_w)])


def kernel(notes, raters, noteEmb, raterEmb, noteBias, raterBias, raterRep,
           globalBias):
    batch = notes.shape[0]
    n_notes = noteEmb.shape[0]
    n_raters = raterEmb.shape[0]
    info = plsc.get_sparse_core_info()
    num_workers = info.num_cores * info.num_subcores
    b_per_w = batch // num_workers

    notes2d = notes.astype(jnp.int32).reshape(batch // CB, CB)
    raters2d = raters.astype(jnp.int32).reshape(batch // CB, CB)
    ne_lin = noteEmb.reshape(n_notes // 8, 8 * N_DIM)
    re_lin = raterEmb.reshape(n_raters // 8, 8 * N_DIM)

    def _pad128(x):
        flat = x.reshape(-1)
        p = (-flat.shape[0]) % 128
        if p:
            flat = jnp.pad(flat, (0, p))
        return flat.reshape(-1, 128)

    nb2 = _pad128(noteBias)
    rb2 = _pad128(raterBias)
    rr2 = _pad128(raterRep)
    gb16 = jnp.broadcast_to(globalBias.astype(jnp.float32), (LANES,))

    mesh = plsc.VectorSubcoreMesh(core_axis_name="c", subcore_axis_name="s")
    run = pl.kernel(
        functools.partial(_mf_kernel, b_per_w, info.num_cores),
        out_type=jax.ShapeDtypeStruct((batch,), jnp.float32),
        mesh=mesh,
        compiler_params=pltpu.CompilerParams(
            needs_layout_passes=False, use_tc_tiling_on_sc=False),
        scratch_types=[
            pltpu.VMEM((b_per_w // CB, CB), jnp.int32),
            pltpu.VMEM((b_per_w // CB, CB), jnp.int32),
            pltpu.VMEM((CB,), jnp.int32),
            pltpu.VMEM((CB,), jnp.int32),
            pltpu.VMEM((CB,), jnp.int32),
            pltpu.VMEM((CB,), jnp.int32),
            pltpu.VMEM((CB, 8 * N_DIM), jnp.float32),
            pltpu.VMEM((CB, 8 * N_DIM), jnp.float32),
            pltpu.VMEM((CB, 128), jnp.float32),
            pltpu.VMEM((CB, 128), jnp.float32),
            pltpu.VMEM((CB, 128), jnp.float32),
            pltpu.VMEM((LANES,), jnp.float32),
            pltpu.VMEM((b_per_w,), jnp.float32),
            pltpu.SemaphoreType.DMA,
        ],
    )
    out = run(notes2d, raters2d, ne_lin, re_lin, nb2, rb2, rr2, gb16)
    return out.reshape(batch, 1)


# speedup vs baseline: 6.1645x; 2.2875x over previous
import functools

import jax
import jax.numpy as jnp
import numpy as np
from jax import lax
from jax.experimental import pallas as pl
from jax.experimental.pallas import tpu as pltpu
from jax.experimental.pallas import tpu_sc as plsc

N_DIM = 16
LANES = 16
CB = 32


def _mf_kernel(b_per_w, num_cores, n_main_blocks, n_tail_base, notes_hbm,
               raters_hbm, ne_s0, ne_s1, ne_tail, re_lin, nb2, rb2, rr2,
               gb_hbm, out_hbm, idx_n, idx_r, qnb, qbn, qr, qbr, nblk0, nblk1,
               tail_v, rrows, nbrows, rbrows, rrrows, gb_v, out_v, sem):
    sid = lax.axis_index("s")
    wid = sid * num_cores + lax.axis_index("c")
    n_chunks = b_per_w // CB
    idx_rows = b_per_w // 128
    row_base = wid * idx_rows

    pltpu.sync_copy(notes_hbm.at[pl.ds(row_base, idx_rows)], idx_n)
    pltpu.sync_copy(raters_hbm.at[pl.ds(row_base, idx_rows)], idx_r)
    pltpu.sync_copy(gb_hbm, gb_v)
    pltpu.sync_copy(ne_tail, tail_v)

    gb = gb_v[...]
    inv_sqrt_dim = np.float32(1.0 / np.sqrt(N_DIM))
    one = jnp.float32(1.0)
    seven = jnp.full((LANES,), 7, jnp.int32)
    c127 = jnp.full((LANES,), 127, jnp.int32)
    three = jnp.full((LANES,), 3, jnp.int32)
    sb7 = jnp.full((LANES,), 7, jnp.int32)
    four = jnp.full((LANES,), 4, jnp.int32)
    max_blk = jnp.full((LANES,), n_main_blocks - 1, jnp.int32)
    tail_base = jnp.full((LANES,), n_tail_base, jnp.int32)
    iota = lax.iota(jnp.int32, LANES)

    for c in range(n_chunks):
        for g in range(CB // LANES):
            b0 = c * CB + g * LANES
            si = pl.ds(b0, LANES)
            so = pl.ds(g * LANES, LANES)
            nidx = idx_n[b0 // 128, pl.ds(b0 % 128, LANES)]
            ridx = idx_r[b0 // 128, pl.ds(b0 % 128, LANES)]
            qnb[so] = lax.min(lax.shift_right_logical(nidx, sb7), max_blk)
            qbn[so] = lax.shift_right_logical(nidx, sb7)
            qr[so] = lax.shift_right_logical(ridx, three)
            qbr[so] = lax.shift_right_logical(ridx, sb7)

        copies = [
            pltpu.make_async_copy(ne_s0.at[qnb], nblk0, sem),
            pltpu.make_async_copy(ne_s1.at[qnb], nblk1, sem),
            pltpu.make_async_copy(re_lin.at[qr], rrows, sem),
            pltpu.make_async_copy(nb2.at[qbn], nbrows, sem),
            pltpu.make_async_copy(rb2.at[qbr], rbrows, sem),
            pltpu.make_async_copy(rr2.at[qbr], rrrows, sem),
        ]
        for cp in copies:
            cp.start()
        for cp in copies:
            cp.wait()

        for g in range(CB // LANES):
            b0 = c * CB + g * LANES
            nidx = idx_n[b0 // 128, pl.ds(b0 % 128, LANES)]
            ridx = idx_r[b0 // 128, pl.ds(b0 % 128, LANES)]
            rows = jnp.full((LANES,), g * LANES, jnp.int32) + iota
            ncol = lax.bitwise_and(nidx, c127)
            rcol = lax.shift_left(lax.bitwise_and(ridx, seven), four)
            is_tail = nidx >= tail_base
            zero16 = jnp.zeros((LANES,), jnp.int32)
            tp0 = lax.shift_left(lax.max(nidx - tail_base, zero16), four)

            acc = jnp.zeros((LANES,), jnp.float32)
            for d in range(N_DIM):
                dv = jnp.full((LANES,), d, jnp.int32)
                blk = nblk0 if d < 8 else nblk1
                rv = jnp.full((LANES,), d % 8, jnp.int32)
                nmain = plsc.load_gather(blk, [rows, rv, ncol])
                tp = tp0 + dv
                ntail = plsc.load_gather(
                    tail_v, [lax.shift_right_logical(tp, sb7),
                             lax.bitwise_and(tp, c127)])
                nval = jnp.where(is_tail, ntail, nmain)
                rval = plsc.load_gather(rrows, [rows, rcol + dv])
                acc = acc + nval * rval
            nb = plsc.load_gather(nbrows, [rows, ncol])
            rb = plsc.load_gather(rbrows, [rows, lax.bitwise_and(ridx, c127)])
            rr = plsc.load_gather(rrrows, [rows, lax.bitwise_and(ridx, c127)])
            pred = acc * inv_sqrt_dim + nb * rr + rb + gb
            out_v[pl.ds(c * CB + g * LANES, LANES)] = one / (one + jnp.exp(-pred))

    pltpu.sync_copy(out_v, out_hbm.at[pl.ds(wid * b_per_w, b_per_w)])


def kernel(notes, raters, noteEmb, raterEmb, noteBias, raterBias, raterRep,
           globalBias):
    batch = notes.shape[0]
    n_notes = noteEmb.shape[0]
    n_raters = raterEmb.shape[0]
    info = plsc.get_sparse_core_info()
    num_workers = info.num_cores * info.num_subcores
    b_per_w = batch // num_workers

    notes2d = notes.astype(jnp.int32).reshape(batch // 128, 128)
    raters2d = raters.astype(jnp.int32).reshape(batch // 128, 128)

    n_main = (n_notes // 128) * 128
    n_blocks = n_main // 128
    ne_s0 = noteEmb[:n_main, 0:8].T.reshape(8, n_blocks, 128).transpose(1, 0, 2)
    ne_s1 = noteEmb[:n_main, 8:16].T.reshape(8, n_blocks, 128).transpose(1, 0, 2)
    ne_tail = noteEmb[n_main:].reshape(8, 128)

    re_lin = raterEmb.reshape(n_raters // 8, 8 * N_DIM)

    def _pad128(x):
        flat = x.reshape(-1)
        p = (-flat.shape[0]) % 128
        if p:
            flat = jnp.pad(flat, (0, p))
        return flat.reshape(-1, 128)

    nb2 = _pad128(noteBias)
    rb2 = _pad128(raterBias)
    rr2 = _pad128(raterRep)
    gb16 = jnp.broadcast_to(globalBias.astype(jnp.float32), (LANES,))

    mesh = plsc.VectorSubcoreMesh(core_axis_name="c", subcore_axis_name="s")
    run = pl.kernel(
        functools.partial(_mf_kernel, b_per_w, info.num_cores, n_blocks,
                          n_main),
        out_type=jax.ShapeDtypeStruct((batch,), jnp.float32),
        mesh=mesh,
        compiler_params=pltpu.CompilerParams(
            needs_layout_passes=False, use_tc_tiling_on_sc=False),
        scratch_types=[
            pltpu.VMEM((b_per_w // 128, 128), jnp.int32),
            pltpu.VMEM((b_per_w // 128, 128), jnp.int32),
            pltpu.VMEM((CB,), jnp.int32),
            pltpu.VMEM((CB,), jnp.int32),
            pltpu.VMEM((CB,), jnp.int32),
            pltpu.VMEM((CB,), jnp.int32),
            pltpu.VMEM((CB, 8, 128), jnp.float32),
            pltpu.VMEM((CB, 8, 128), jnp.float32),
            pltpu.VMEM((8, 128), jnp.float32),
            pltpu.VMEM((CB, 128), jnp.float32),
            pltpu.VMEM((CB, 128), jnp.float32),
            pltpu.VMEM((CB, 128), jnp.float32),
            pltpu.VMEM((CB, 128), jnp.float32),
            pltpu.VMEM((LANES,), jnp.float32),
            pltpu.VMEM((b_per_w,), jnp.float32),
            pltpu.SemaphoreType.DMA,
        ],
    )
    out = run(notes2d, raters2d, ne_s0, ne_s1, ne_tail, re_lin, nb2, rb2, rr2,
              gb16)
    return out.reshape(batch, 1)


# speedup vs baseline: 6.2333x; 1.0112x over previous
import functools

import jax
import jax.numpy as jnp
import numpy as np
from jax import lax
from jax.experimental import pallas as pl
from jax.experimental.pallas import tpu as pltpu
from jax.experimental.pallas import tpu_sc as plsc

N_DIM = 16
LANES = 16
CA = 128
CB = 32


def _rater_kernel(b_per_w, num_cores, raters_hbm, re_lin, nb2, rb2, rr2,
                  gb_hbm, notes_hbm, stage_hbm, part_hbm, idx_r, idx_n, qr,
                  qbr, qbn, rrows, nbrows, rbrows, rrrows, gb_v, rst, part_v,
                  sem):
    sid = lax.axis_index("s")
    wid = sid * num_cores + lax.axis_index("c")
    n_chunks = b_per_w // CA
    row_base = wid * n_chunks

    pltpu.sync_copy(raters_hbm.at[pl.ds(row_base, n_chunks)], idx_r)
    pltpu.sync_copy(notes_hbm.at[pl.ds(row_base, n_chunks)], idx_n)
    pltpu.sync_copy(gb_hbm, gb_v)

    gb = gb_v[...]
    seven = jnp.full((LANES,), 7, jnp.int32)
    c127 = jnp.full((LANES,), 127, jnp.int32)
    three = jnp.full((LANES,), 3, jnp.int32)
    sb7 = jnp.full((LANES,), 7, jnp.int32)
    four = jnp.full((LANES,), 4, jnp.int32)
    iota = lax.iota(jnp.int32, LANES)

    for c in range(n_chunks):
        for g in range(CA // LANES):
            so = pl.ds(g * LANES, LANES)
            ridx = idx_r[c, so]
            nidx = idx_n[c, so]
            qr[so] = lax.shift_right_logical(ridx, three)
            qbr[so] = lax.shift_right_logical(ridx, sb7)
            qbn[so] = lax.shift_right_logical(nidx, sb7)

        copies = [
            pltpu.make_async_copy(re_lin.at[qr], rrows, sem),
            pltpu.make_async_copy(nb2.at[qbn], nbrows, sem),
            pltpu.make_async_copy(rb2.at[qbr], rbrows, sem),
            pltpu.make_async_copy(rr2.at[qbr], rrrows, sem),
        ]
        for cp in copies:
            cp.start()
        for cp in copies:
            cp.wait()

        for g in range(CA // LANES):
            so = pl.ds(g * LANES, LANES)
            b0 = c * CA + g * LANES
            sbuf = pl.ds(b0, LANES)
            ridx = idx_r[c, so]
            nidx = idx_n[c, so]
            rows = jnp.full((LANES,), g * LANES, jnp.int32) + iota
            rcol = lax.shift_left(lax.bitwise_and(ridx, seven), four)
            for d in range(N_DIM):
                dv = jnp.full((LANES,), d, jnp.int32)
                rst[d, sbuf] = plsc.load_gather(rrows, [rows, rcol + dv])
            nb = plsc.load_gather(nbrows, [rows, lax.bitwise_and(nidx, c127)])
            rb = plsc.load_gather(rbrows, [rows, lax.bitwise_and(ridx, c127)])
            rr = plsc.load_gather(rrrows, [rows, lax.bitwise_and(ridx, c127)])
            part_v[sbuf] = nb * rr + rb + gb

    pltpu.sync_copy(rst, stage_hbm.at[:, pl.ds(wid * b_per_w, b_per_w)])
    pltpu.sync_copy(part_v, part_hbm.at[pl.ds(wid * b_per_w, b_per_w)])


def _note_kernel(b_per_w, num_cores, n_main_blocks, n_tail_base, notes_hbm,
                 ne_s0, ne_s1, ne_tail, stage_hbm, part_hbm, out_hbm, idx_n,
                 qnb, nblk0, nblk1, tail_v, rst, part_v, out_v, sem):
    sid = lax.axis_index("s")
    wid = sid * num_cores + lax.axis_index("c")
    n_chunks = b_per_w // CB
    idx_rows = b_per_w // 128
    row_base = wid * idx_rows

    pltpu.sync_copy(notes_hbm.at[pl.ds(row_base, idx_rows)], idx_n)
    pltpu.sync_copy(ne_tail, tail_v)
    pltpu.sync_copy(stage_hbm.at[:, pl.ds(wid * b_per_w, b_per_w)], rst)
    pltpu.sync_copy(part_hbm.at[pl.ds(wid * b_per_w, b_per_w)], part_v)

    inv_sqrt_dim = np.float32(1.0 / np.sqrt(N_DIM))
    one = jnp.float32(1.0)
    c127 = jnp.full((LANES,), 127, jnp.int32)
    sb7 = jnp.full((LANES,), 7, jnp.int32)
    four = jnp.full((LANES,), 4, jnp.int32)
    max_blk = jnp.full((LANES,), n_main_blocks - 1, jnp.int32)
    tail_base = jnp.full((LANES,), n_tail_base, jnp.int32)
    zero16 = jnp.zeros((LANES,), jnp.int32)
    iota = lax.iota(jnp.int32, LANES)

    for c in range(n_chunks):
        for g in range(CB // LANES):
            b0 = c * CB + g * LANES
            so = pl.ds(g * LANES, LANES)
            nidx = idx_n[b0 // 128, pl.ds(b0 % 128, LANES)]
            qnb[so] = lax.min(lax.shift_right_logical(nidx, sb7), max_blk)

        copies = [
            pltpu.make_async_copy(ne_s0.at[qnb], nblk0, sem),
            pltpu.make_async_copy(ne_s1.at[qnb], nblk1, sem),
        ]
        for cp in copies:
            cp.start()
        for cp in copies:
            cp.wait()

        for g in range(CB // LANES):
            b0 = c * CB + g * LANES
            sbuf = pl.ds(b0, LANES)
            nidx = idx_n[b0 // 128, pl.ds(b0 % 128, LANES)]
            rows = jnp.full((LANES,), g * LANES, jnp.int32) + iota
            ncol = lax.bitwise_and(nidx, c127)
            is_tail = nidx >= tail_base
            tp0 = lax.shift_left(lax.max(nidx - tail_base, zero16), four)

            acc = jnp.zeros((LANES,), jnp.float32)
            for d in range(N_DIM):
                dv = jnp.full((LANES,), d, jnp.int32)
                blk = nblk0 if d < 8 else nblk1
                rv = jnp.full((LANES,), d % 8, jnp.int32)
                nmain = plsc.load_gather(blk, [rows, rv, ncol])
                tp = tp0 + dv
                ntail = plsc.load_gather(
                    tail_v, [lax.shift_right_logical(tp, sb7),
                             lax.bitwise_and(tp, c127)])
                nval = jnp.where(is_tail, ntail, nmain)
                acc = acc + nval * rst[d, sbuf]
            pred = acc * inv_sqrt_dim + part_v[sbuf]
            out_v[sbuf] = one / (one + jnp.exp(-pred))

    pltpu.sync_copy(out_v, out_hbm.at[pl.ds(wid * b_per_w, b_per_w)])


def kernel(notes, raters, noteEmb, raterEmb, noteBias, raterBias, raterRep,
           globalBias):
    batch = notes.shape[0]
    n_notes = noteEmb.shape[0]
    n_raters = raterEmb.shape[0]
    info = plsc.get_sparse_core_info()
    num_workers = info.num_cores * info.num_subcores
    b_per_w = batch // num_workers

    notes2d = notes.astype(jnp.int32).reshape(batch // 128, 128)
    raters2d = raters.astype(jnp.int32).reshape(batch // 128, 128)

    n_main = (n_notes // 128) * 128
    n_blocks = n_main // 128
    ne_s0 = noteEmb[:n_main, 0:8].T.reshape(8, n_blocks, 128).transpose(1, 0, 2)
    ne_s1 = noteEmb[:n_main, 8:16].T.reshape(8, n_blocks, 128).transpose(1, 0, 2)
    ne_tail = noteEmb[n_main:].reshape(8, 128)

    re_lin = raterEmb.reshape(n_raters // 8, 8 * N_DIM)

    def _pad128(x):
        flat = x.reshape(-1)
        p = (-flat.shape[0]) % 128
        if p:
            flat = jnp.pad(flat, (0, p))
        return flat.reshape(-1, 128)

    nb2 = _pad128(noteBias)
    rb2 = _pad128(raterBias)
    rr2 = _pad128(raterRep)
    gb16 = jnp.broadcast_to(globalBias.astype(jnp.float32), (LANES,))

    mesh = plsc.VectorSubcoreMesh(core_axis_name="c", subcore_axis_name="s")
    sc_params = pltpu.CompilerParams(
        needs_layout_passes=False, use_tc_tiling_on_sc=False)

    run_a = pl.kernel(
        functools.partial(_rater_kernel, b_per_w, info.num_cores),
        out_type=(
            jax.ShapeDtypeStruct((N_DIM, batch), jnp.float32),
            jax.ShapeDtypeStruct((batch,), jnp.float32),
        ),
        mesh=mesh,
        compiler_params=sc_params,
        scratch_types=[
            pltpu.VMEM((b_per_w // CA, CA), jnp.int32),
            pltpu.VMEM((b_per_w // CA, CA), jnp.int32),
            pltpu.VMEM((CA,), jnp.int32),
            pltpu.VMEM((CA,), jnp.int32),
            pltpu.VMEM((CA,), jnp.int32),
            pltpu.VMEM((CA, 128), jnp.float32),
            pltpu.VMEM((CA, 128), jnp.float32),
            pltpu.VMEM((CA, 128), jnp.float32),
            pltpu.VMEM((CA, 128), jnp.float32),
            pltpu.VMEM((LANES,), jnp.float32),
            pltpu.VMEM((N_DIM, b_per_w), jnp.float32),
            pltpu.VMEM((b_per_w,), jnp.float32),
            pltpu.SemaphoreType.DMA,
        ],
    )
    stage, part = run_a(raters2d, re_lin, nb2, rb2, rr2, gb16, notes2d)

    run_b = pl.kernel(
        functools.partial(_note_kernel, b_per_w, info.num_cores, n_blocks,
                          n_main),
        out_type=jax.ShapeDtypeStruct((batch,), jnp.float32),
        mesh=mesh,
        compiler_params=sc_params,
        scratch_types=[
            pltpu.VMEM((b_per_w // 128, 128), jnp.int32),
            pltpu.VMEM((CB,), jnp.int32),
            pltpu.VMEM((CB, 8, 128), jnp.float32),
            pltpu.VMEM((CB, 8, 128), jnp.float32),
            pltpu.VMEM((8, 128), jnp.float32),
            pltpu.VMEM((N_DIM, b_per_w), jnp.float32),
            pltpu.VMEM((b_per_w,), jnp.float32),
            pltpu.VMEM((b_per_w,), jnp.float32),
            pltpu.SemaphoreType.DMA,
        ],
    )
    out = run_b(notes2d, ne_s0, ne_s1, ne_tail, stage, part)
    return out.reshape(batch, 1)
